# baseline (speedup 1.0000x reference)
import functools

import jax
import jax.numpy as jnp
from jax import lax
from jax.experimental import pallas as pl
from jax.experimental.pallas import tpu as pltpu
from jax.experimental.pallas import tpu_sc as plsc

N_NODES = 10000
N_HE = 10000
N_EDGES = 160000
NT = 16
NCORE = 2
C = 128
EPT = N_EDGES // NT
ECH = 80
EPAD = ECH * 128 - EPT
EPT2 = N_EDGES // (NCORE * NT)
ECH2 = 40
EPAD2 = ECH2 * 128 - EPT2
ACC_ROWS = 10112
SENT = N_NODES
BN = 1000


def _tile_pad(idx, sentinel):
    a = idx.reshape(NT, EPT)
    a = jnp.pad(a, ((0, 0), (0, EPAD)), constant_values=sentinel)
    return a.reshape(NT, ECH, 128)


def _make_gather_idx(idx_pad, nch):
    nc = NCORE * nch
    shift = (jnp.arange(nc, dtype=jnp.int32) * N_NODES)[:, None, None, None]
    g = idx_pad[None] + shift
    g = g.reshape(nc, NT, 2, 40, 128)
    g = jnp.pad(g, ((0, 0), (0, 0), (0, 0), (0, 1), (0, 0)))
    g = g.reshape(NCORE, nch, NT, 2, 41, 128)
    return g.transpose(0, 2, 1, 3, 4, 5)


def _tile_pad2(idx, sentinel):
    a = idx.reshape(NCORE, NT, EPT2)
    a = jnp.pad(a, ((0, 0), (0, 0), (0, EPAD2)), constant_values=sentinel)
    return a.reshape(NCORE, NT, ECH2, 128)


def _make_gather_idx2(idx_pad2, nch):
    shift = (jnp.arange(nch, dtype=jnp.int32) * N_NODES).reshape(1, 1, nch, 1, 1)
    g = idx_pad2[:, :, None] + shift
    g = jnp.pad(g, ((0, 0), (0, 0), (0, 0), (0, 1), (0, 0)))
    return g[:, :, :, None]


def _sc_pass(full):
    nh = 2 if full else 1
    mesh = plsc.VectorSubcoreMesh(core_axis_name="c", subcore_axis_name="s")
    out_t = (jax.ShapeDtypeStruct((6, N_NODES, C), jnp.float32) if full
             else jax.ShapeDtypeStruct((NCORE, 3, N_NODES, C), jnp.float32))

    @functools.partial(
        pl.kernel, mesh=mesh,
        out_type=out_t,
        scratch_types=[
            pltpu.VMEM((41, 128), jnp.int32),
            pltpu.VMEM((nh * 40, 128), jnp.int32),
            pltpu.VMEM((128, C), jnp.float32),
            pltpu.VMEM((128, C), jnp.float32),
            pltpu.VMEM_SHARED((ACC_ROWS, C), jnp.float32),
            pltpu.SemaphoreType.DMA,
            pltpu.SemaphoreType.DMA,
        ],
    )
    def k(src, gidx, sidx, zeros, out, gidx_v, sidx_v, rows_a, rows_b, acc,
          sem_a, sem_b):
        core = lax.axis_index("c")
        s = lax.axis_index("s")
        if full:
            pltpu.sync_copy(sidx.at[s], sidx_v)
        else:
            pltpu.sync_copy(sidx.at[core, s], sidx_v)
        for cc in range(3):
            pltpu.sync_copy(zeros, acc.at[pl.ds(s * 632, 632)])
            plsc.subcore_barrier()
            for h in range(nh):
                pltpu.sync_copy(gidx.at[core, s, cc, h], gidx_v)
                pltpu.async_copy(src.at[gidx_v.at[0]], rows_a, sem_a)

                def pair(i, carry):
                    ja = 2 * i
                    pltpu.make_async_copy(
                        src.at[gidx_v.at[ja]], rows_a, sem_a).wait()
                    pltpu.async_copy(src.at[gidx_v.at[ja + 1]], rows_b, sem_b)
                    pltpu.sync_copy(rows_a, acc.at[sidx_v.at[h * 40 + ja]],
                                    add=True)
                    pltpu.make_async_copy(
                        src.at[gidx_v.at[ja + 1]], rows_b, sem_b).wait()
                    pltpu.async_copy(src.at[gidx_v.at[ja + 2]], rows_a, sem_a)
                    pltpu.sync_copy(rows_b, acc.at[sidx_v.at[h * 40 + ja + 1]],
                                    add=True)
                    return carry

                lax.fori_loop(0, 20, pair, 0)
                pltpu.make_async_copy(src.at[gidx_v.at[40]], rows_a,
                                      sem_a).wait()
            plsc.subcore_barrier()
            if full:
                chunk = core * 3 + cc
                pltpu.sync_copy(acc.at[pl.ds(s * 624, 640)],
                                out.at[chunk, pl.ds(s * 624, 640)])
            else:
                pltpu.sync_copy(acc.at[pl.ds(s * 624, 640)],
                                out.at[core, cc, pl.ds(s * 624, 640)])
            plsc.subcore_barrier()

    return k


def _sc_degrees():
    mesh = plsc.VectorSubcoreMesh(core_axis_name="c", subcore_axis_name="s")

    @functools.partial(
        pl.kernel, mesh=mesh,
        out_type=jax.ShapeDtypeStruct((2, N_NODES, C), jnp.float32),
        scratch_types=[
            pltpu.VMEM((ECH, 128), jnp.int32),
            pltpu.VMEM((128, C), jnp.float32),
            pltpu.VMEM_SHARED((ACC_ROWS, C), jnp.float32),
        ],
    )
    def k(sidx2, ones, zeros, out, sidx_v, ones_v, acc):
        core = lax.axis_index("c")
        s = lax.axis_index("s")
        pltpu.sync_copy(sidx2.at[core, s], sidx_v)
        pltpu.sync_copy(ones, ones_v)
        pltpu.sync_copy(zeros, acc.at[pl.ds(s * 632, 632)])
        plsc.subcore_barrier()

        def body(j, carry):
            pltpu.sync_copy(ones_v, acc.at[sidx_v.at[j]], add=True)
            return carry

        lax.fori_loop(0, ECH, body, 0)
        plsc.subcore_barrier()
        pltpu.sync_copy(acc.at[pl.ds(s * 624, 640)],
                        out.at[core, pl.ds(s * 624, 640)])

    return k


def _load_raw(r_ref, parts):
    return (r_ref[0, 0] + r_ref[1, 0]) if parts else r_ref[0]


def _raw_spec(parts, imap3):
    if parts:
        return pl.BlockSpec((2, 1, BN, C), lambda *g: (0,) + imap3(*g))
    return pl.BlockSpec((1, BN, C), imap3)


def _mm_in_flat(x, w, nco):
    din = x.shape[1]

    def body(x_ref, w_ref, o_ref):
        o_ref[...] = jnp.dot(x_ref[...], w_ref[...],
                             preferred_element_type=jnp.float32)[None]

    return pl.pallas_call(
        body,
        grid=(nco, N_NODES // BN),
        in_specs=[
            pl.BlockSpec((BN, din), lambda o, r: (r, 0)),
            pl.BlockSpec((din, C), lambda o, r: (0, o)),
        ],
        out_specs=pl.BlockSpec((1, BN, C), lambda o, r: (o, r, 0)),
        out_shape=jax.ShapeDtypeStruct((nco, N_NODES, C), jnp.float32),
    )(x, w)


def _scale_rows(raw, cnt, parts):
    nc = raw.shape[1] if parts else raw.shape[0]

    def body(r_ref, c_ref, o_ref):
        c = c_ref[:, 0:1]
        inv = jnp.where(c > 0, 1.0 / c, 0.0)
        o_ref[...] = (_load_raw(r_ref, parts) * inv)[None]

    return pl.pallas_call(
        body,
        grid=(nc, N_NODES // BN),
        in_specs=[
            _raw_spec(parts, lambda o, r: (o, r, 0)),
            pl.BlockSpec((BN, C), lambda o, r: (r, 0)),
        ],
        out_specs=pl.BlockSpec((1, BN, C), lambda o, r: (o, r, 0)),
        out_shape=jax.ShapeDtypeStruct((nc, N_NODES, C), jnp.float32),
    )(raw, cnt)


def _stats(raw, cnt, parts):
    nc = raw.shape[1] if parts else raw.shape[0]

    def body(r_ref, c_ref, s_ref, q_ref):
        r = pl.program_id(1)
        c = c_ref[:, 0:1]
        inv = jnp.where(c > 0, 1.0 / c, 0.0)
        y = _load_raw(r_ref, parts) * inv
        s1 = jnp.broadcast_to(jnp.sum(y, axis=0, keepdims=True), (8, C))[None]
        q1 = jnp.broadcast_to(jnp.sum(y * y, axis=0, keepdims=True), (8, C))[None]

        @pl.when(r == 0)
        def _():
            s_ref[...] = s1
            q_ref[...] = q1

        @pl.when(r != 0)
        def _():
            s_ref[...] += s1
            q_ref[...] += q1

    return pl.pallas_call(
        body,
        grid=(nc, N_NODES // BN),
        in_specs=[
            _raw_spec(parts, lambda o, r: (o, r, 0)),
            pl.BlockSpec((BN, C), lambda o, r: (r, 0)),
        ],
        out_specs=[
            pl.BlockSpec((1, 8, C), lambda o, r: (o, 0, 0)),
            pl.BlockSpec((1, 8, C), lambda o, r: (o, 0, 0)),
        ],
        out_shape=[
            jax.ShapeDtypeStruct((nc, 8, C), jnp.float32),
            jax.ShapeDtypeStruct((nc, 8, C), jnp.float32),
        ],
    )(raw, cnt)


def _bn_z(r_ref, c_ref, g_ref, bt_ref, s_ref, q_ref, parts):
    m = s_ref[0, 0:1, :] * (1.0 / N_NODES)
    msq = q_ref[0, 0:1, :] * (1.0 / N_NODES)
    inv_std = lax.rsqrt(jnp.maximum(msq - m * m, 0.0) + 1e-5)
    c = c_ref[:, 0:1]
    dinv = jnp.where(c > 0, 1.0 / c, 0.0)
    y = _load_raw(r_ref, parts) * dinv
    return jnp.maximum((y - m) * inv_std * g_ref[0] + bt_ref[0], 0.0)


def _bn_mm(raw, cnt, g, bt, s, q, wc, parts, x0c=None):
    nci, nco = wc.shape[0], wc.shape[1]
    has_res = x0c is not None

    def body(*refs):
        if has_res:
            r_ref, c_ref, g_ref, bt_ref, s_ref, q_ref, x0_ref, w_ref, o_ref = refs
        else:
            r_ref, c_ref, g_ref, bt_ref, s_ref, q_ref, w_ref, o_ref = refs
        kk = pl.program_id(2)
        z = _bn_z(r_ref, c_ref, g_ref, bt_ref, s_ref, q_ref, parts)
        if has_res:
            z = z + x0_ref[0]
        acc = jnp.dot(z, w_ref[0, 0], preferred_element_type=jnp.float32)[None]

        @pl.when(kk == 0)
        def _():
            o_ref[...] = acc

        @pl.when(kk != 0)
        def _():
            o_ref[...] += acc

    in_specs = [
        _raw_spec(parts, lambda o, r, kk: (kk, r, 0)),
        pl.BlockSpec((BN, C), lambda o, r, kk: (r, 0)),
        pl.BlockSpec((1, 1, C), lambda o, r, kk: (kk, 0, 0)),
        pl.BlockSpec((1, 1, C), lambda o, r, kk: (kk, 0, 0)),
        pl.BlockSpec((1, 8, C), lambda o, r, kk: (kk, 0, 0)),
        pl.BlockSpec((1, 8, C), lambda o, r, kk: (kk, 0, 0)),
    ]
    args = [raw, cnt, g.reshape(nci, 1, C), bt.reshape(nci, 1, C), s, q]
    if has_res:
        in_specs.append(pl.BlockSpec((1, BN, C), lambda o, r, kk: (kk, r, 0)))
        args.append(x0c)
    in_specs.append(pl.BlockSpec((1, 1, C, C), lambda o, r, kk: (kk, o, 0, 0)))
    args.append(wc)

    return pl.pallas_call(
        body,
        grid=(nco, N_NODES // BN, nci),
        in_specs=in_specs,
        out_specs=pl.BlockSpec((1, BN, C), lambda o, r, kk: (o, r, 0)),
        out_shape=jax.ShapeDtypeStruct((nco, N_NODES, C), jnp.float32),
    )(*args)


def _bn_final(raw, cnt, g, bt, s, q, parts):
    nc = raw.shape[1] if parts else raw.shape[0]

    def body(r_ref, c_ref, g_ref, bt_ref, s_ref, q_ref, o_ref):
        o_ref[...] = _bn_z(r_ref, c_ref, g_ref, bt_ref, s_ref, q_ref, parts)[None]

    return pl.pallas_call(
        body,
        grid=(nc, N_NODES // BN),
        in_specs=[
            _raw_spec(parts, lambda o, r: (o, r, 0)),
            pl.BlockSpec((BN, C), lambda o, r: (r, 0)),
            pl.BlockSpec((1, 1, C), lambda o, r: (o, 0, 0)),
            pl.BlockSpec((1, 1, C), lambda o, r: (o, 0, 0)),
            pl.BlockSpec((1, 8, C), lambda o, r: (o, 0, 0)),
            pl.BlockSpec((1, 8, C), lambda o, r: (o, 0, 0)),
        ],
        out_specs=pl.BlockSpec((1, BN, C), lambda o, r: (o, r, 0)),
        out_shape=jax.ShapeDtypeStruct((nc, N_NODES, C), jnp.float32),
    )(raw, cnt, g.reshape(nc, 1, C), bt.reshape(nc, 1, C), s, q)


def _chunk_w(w):
    di, do = w.shape
    nci, nco = di // C, do // C
    return w.reshape(nci, C, nco, C).transpose(0, 2, 1, 3)


def kernel(x, edge, W1, b1, g1, bt1, W2, b2, g2, bt2, W3, b3, g3, bt3,
           W4, b4, g4, bt4):
    nidx = edge[0]
    hidx = edge[1]

    nid_s = _tile_pad(nidx, SENT)
    hid_s = _tile_pad(hidx, SENT)
    nid_g = _make_gather_idx(_tile_pad(nidx, 0), 3)
    hid_g = _make_gather_idx(_tile_pad(hidx, 0), 3)
    nid_s2 = _tile_pad2(nidx, SENT)
    hid_s2 = _tile_pad2(hidx, SENT)
    nid_g2 = _make_gather_idx2(_tile_pad2(nidx, 0), 3)
    hid_g2 = _make_gather_idx2(_tile_pad2(hidx, 0), 3)

    ones = jnp.ones((128, C), jnp.float32)
    zeros = jnp.zeros((632, C), jnp.float32)

    sidx2 = jnp.stack([nid_s, hid_s])
    cnts = _sc_degrees()(sidx2, ones, zeros)
    d16 = cnts[0]
    bd16 = cnts[1]

    pass_full = _sc_pass(True)
    pass_half = _sc_pass(False)

    def conv6(xw):
        he_raw = pass_full(xw.reshape(6 * N_NODES, C), nid_g, hid_s, zeros)
        he_s = _scale_rows(he_raw, bd16, False)
        return pass_full(he_s.reshape(6 * N_NODES, C), hid_g, nid_s, zeros)

    def conv3(xw):
        he_raw = pass_half(xw.reshape(3 * N_NODES, C), nid_g2, hid_s2, zeros)
        he_s = _scale_rows(he_raw, bd16, True)
        return pass_half(he_s.reshape(3 * N_NODES, C), hid_g2, nid_s2, zeros)

    xw = _mm_in_flat(x, W1, 6)
    r1 = conv6(xw)
    s1, q1 = _stats(r1, d16, False)
    xw = _bn_mm(r1, d16, g1, bt1, s1, q1, _chunk_w(W2), False)
    r2 = conv6(xw)
    s2, q2 = _stats(r2, d16, False)
    xw = _bn_mm(r2, d16, g2, bt2, s2, q2, _chunk_w(W3), False)
    r3 = conv3(xw)
    s3, q3 = _stats(r3, d16, True)
    x0c = x.reshape(N_NODES, 3, C).transpose(1, 0, 2)
    xw = _bn_mm(r3, d16, g3, bt3, s3, q3, _chunk_w(W4), True, x0c=x0c)
    r4 = conv3(xw)
    s4, q4 = _stats(r4, d16, True)
    h = _bn_final(r4, d16, g4, bt4, s4, q4, True)
    return h.transpose(1, 0, 2).reshape(N_NODES, 3 * C)


# speedup vs baseline: 1.6064x; 1.6064x over previous
import functools

import jax
import jax.numpy as jnp
from jax import lax
from jax.experimental import pallas as pl
from jax.experimental.pallas import tpu as pltpu
from jax.experimental.pallas import tpu_sc as plsc

N_NODES = 10000
N_HE = 10000
N_EDGES = 160000
NT = 16
NCORE = 2
C = 128
EPT = N_EDGES // NT
ECH = 80
EPAD = ECH * 128 - EPT
EPT2 = N_EDGES // (NCORE * NT)
ECH2 = 40
EPAD2 = ECH2 * 128 - EPT2
ACC_ROWS = 10112
SENT = N_NODES
BN = 1000


def _tile_pad(idx, sentinel):
    a = idx.reshape(NT, EPT)
    a = jnp.pad(a, ((0, 0), (0, EPAD)), constant_values=sentinel)
    return a.reshape(NT, ECH, 128)


def _make_gather_idx(idx_pad, nch):
    nc = NCORE * nch
    shift = (jnp.arange(nc, dtype=jnp.int32) * N_NODES)[:, None, None, None]
    g = idx_pad[None] + shift
    g = g.reshape(nc, NT, 2, 40, 128)
    g = jnp.pad(g, ((0, 0), (0, 0), (0, 0), (0, 1), (0, 0)))
    g = g.reshape(NCORE, nch, NT, 2, 41, 128)
    return g.transpose(0, 2, 1, 3, 4, 5)


def _tile_pad2(idx, sentinel):
    a = idx.reshape(NCORE, NT, EPT2)
    a = jnp.pad(a, ((0, 0), (0, 0), (0, EPAD2)), constant_values=sentinel)
    return a.reshape(NCORE, NT, ECH2, 128)


def _make_gather_idx2(idx_pad2, nch):
    shift = (jnp.arange(nch, dtype=jnp.int32) * N_NODES).reshape(1, 1, nch, 1, 1)
    g = idx_pad2[:, :, None] + shift
    g = jnp.pad(g, ((0, 0), (0, 0), (0, 0), (0, 1), (0, 0)))
    return g[:, :, :, None]


def _sc_pass(full):
    nh = 2 if full else 1
    mesh = plsc.VectorSubcoreMesh(core_axis_name="c", subcore_axis_name="s")
    out_t = (jax.ShapeDtypeStruct((6, N_NODES, C), jnp.float32) if full
             else jax.ShapeDtypeStruct((NCORE, 3, N_NODES, C), jnp.float32))

    @functools.partial(
        pl.kernel, mesh=mesh,
        out_type=out_t,
        scratch_types=[
            pltpu.VMEM((41, 128), jnp.int32),
            pltpu.VMEM((40, 128), jnp.int32),
            pltpu.VMEM((128, C), jnp.float32),
            pltpu.VMEM((128, C), jnp.float32),
            pltpu.VMEM((40, C), jnp.float32),
            pltpu.VMEM_SHARED((ACC_ROWS, C), jnp.float32),
            pltpu.SemaphoreType.DMA,
            pltpu.SemaphoreType.DMA,
        ],
    )
    def k(src, gidx, sidx, zeros, out, gidx_v, sidx_v, rows_a, rows_b, zero_v,
          acc, sem_a, sem_b):
        core = lax.axis_index("c")
        s = lax.axis_index("s")
        pltpu.sync_copy(zeros, zero_v)
        if not full:
            pltpu.sync_copy(sidx.at[core, s], sidx_v)
        for cc in range(3):
            for z in range(15):
                pltpu.sync_copy(zero_v, acc.at[pl.ds(s * 632 + z * 40, 40)])
            pltpu.sync_copy(zero_v.at[pl.ds(0, 32)],
                            acc.at[pl.ds(s * 632 + 600, 32)])
            plsc.subcore_barrier()
            for h in range(nh):
                pltpu.sync_copy(gidx.at[core, s, cc, h], gidx_v)
                if full:
                    pltpu.sync_copy(sidx.at[s, pl.ds(h * 40, 40)], sidx_v)

                def pair(i, carry):
                    ja = 2 * i
                    cpa = pltpu.async_copy(src.at[gidx_v.at[ja]], rows_a,
                                           sem_a)
                    cpb = pltpu.async_copy(src.at[gidx_v.at[ja + 1]], rows_b,
                                           sem_b)
                    cpa.wait()
                    pltpu.sync_copy(rows_a, acc.at[sidx_v.at[ja]], add=True)
                    cpb.wait()
                    pltpu.sync_copy(rows_b, acc.at[sidx_v.at[ja + 1]],
                                    add=True)
                    return carry

                lax.fori_loop(0, 20, pair, 0)
            plsc.subcore_barrier()
            if full:
                chunk = core * 3 + cc
                pltpu.sync_copy(acc.at[pl.ds(s * 624, 640)],
                                out.at[chunk, pl.ds(s * 624, 640)])
            else:
                pltpu.sync_copy(acc.at[pl.ds(s * 624, 640)],
                                out.at[core, cc, pl.ds(s * 624, 640)])
            plsc.subcore_barrier()

    return k


def _sc_degrees():
    mesh = plsc.VectorSubcoreMesh(core_axis_name="c", subcore_axis_name="s")

    @functools.partial(
        pl.kernel, mesh=mesh,
        out_type=jax.ShapeDtypeStruct((2, N_NODES, C), jnp.float32),
        scratch_types=[
            pltpu.VMEM((ECH, 128), jnp.int32),
            pltpu.VMEM((128, C), jnp.float32),
            pltpu.VMEM((40, C), jnp.float32),
            pltpu.VMEM_SHARED((ACC_ROWS, C), jnp.float32),
        ],
    )
    def k(sidx2, ones, zeros, out, sidx_v, ones_v, zero_v, acc):
        core = lax.axis_index("c")
        s = lax.axis_index("s")
        pltpu.sync_copy(sidx2.at[core, s], sidx_v)
        pltpu.sync_copy(ones, ones_v)
        pltpu.sync_copy(zeros, zero_v)
        for z in range(15):
            pltpu.sync_copy(zero_v, acc.at[pl.ds(s * 632 + z * 40, 40)])
        pltpu.sync_copy(zero_v.at[pl.ds(0, 32)],
                        acc.at[pl.ds(s * 632 + 600, 32)])
        plsc.subcore_barrier()

        def body(j, carry):
            pltpu.sync_copy(ones_v, acc.at[sidx_v.at[j]], add=True)
            return carry

        lax.fori_loop(0, ECH, body, 0)
        plsc.subcore_barrier()
        pltpu.sync_copy(acc.at[pl.ds(s * 624, 640)],
                        out.at[core, pl.ds(s * 624, 640)])

    return k


def _load_raw(r_ref, parts):
    return (r_ref[0, 0] + r_ref[1, 0]) if parts else r_ref[0]


def _raw_spec(parts, imap3):
    if parts:
        return pl.BlockSpec((2, 1, BN, C), lambda *g: (0,) + imap3(*g))
    return pl.BlockSpec((1, BN, C), imap3)


def _mm_in_flat(x, w, nco):
    din = x.shape[1]

    def body(x_ref, w_ref, o_ref):
        o_ref[...] = jnp.dot(x_ref[...], w_ref[...],
                             preferred_element_type=jnp.float32)[None]

    return pl.pallas_call(
        body,
        grid=(nco, N_NODES // BN),
        in_specs=[
            pl.BlockSpec((BN, din), lambda o, r: (r, 0)),
            pl.BlockSpec((din, C), lambda o, r: (0, o)),
        ],
        out_specs=pl.BlockSpec((1, BN, C), lambda o, r: (o, r, 0)),
        out_shape=jax.ShapeDtypeStruct((nco, N_NODES, C), jnp.float32),
    )(x, w)


def _scale_rows(raw, cnt, parts):
    nc = raw.shape[1] if parts else raw.shape[0]

    def body(r_ref, c_ref, o_ref):
        c = c_ref[:, 0:1]
        inv = jnp.where(c > 0, 1.0 / c, 0.0)
        o_ref[...] = (_load_raw(r_ref, parts) * inv)[None]

    return pl.pallas_call(
        body,
        grid=(nc, N_NODES // BN),
        in_specs=[
            _raw_spec(parts, lambda o, r: (o, r, 0)),
            pl.BlockSpec((BN, C), lambda o, r: (r, 0)),
        ],
        out_specs=pl.BlockSpec((1, BN, C), lambda o, r: (o, r, 0)),
        out_shape=jax.ShapeDtypeStruct((nc, N_NODES, C), jnp.float32),
    )(raw, cnt)


def _stats(raw, cnt, parts):
    nc = raw.shape[1] if parts else raw.shape[0]

    def body(r_ref, c_ref, s_ref, q_ref):
        r = pl.program_id(1)
        c = c_ref[:, 0:1]
        inv = jnp.where(c > 0, 1.0 / c, 0.0)
        y = _load_raw(r_ref, parts) * inv
        s1 = jnp.broadcast_to(jnp.sum(y, axis=0, keepdims=True), (8, C))[None]
        q1 = jnp.broadcast_to(jnp.sum(y * y, axis=0, keepdims=True), (8, C))[None]

        @pl.when(r == 0)
        def _():
            s_ref[...] = s1
            q_ref[...] = q1

        @pl.when(r != 0)
        def _():
            s_ref[...] += s1
            q_ref[...] += q1

    return pl.pallas_call(
        body,
        grid=(nc, N_NODES // BN),
        in_specs=[
            _raw_spec(parts, lambda o, r: (o, r, 0)),
            pl.BlockSpec((BN, C), lambda o, r: (r, 0)),
        ],
        out_specs=[
            pl.BlockSpec((1, 8, C), lambda o, r: (o, 0, 0)),
            pl.BlockSpec((1, 8, C), lambda o, r: (o, 0, 0)),
        ],
        out_shape=[
            jax.ShapeDtypeStruct((nc, 8, C), jnp.float32),
            jax.ShapeDtypeStruct((nc, 8, C), jnp.float32),
        ],
    )(raw, cnt)


def _bn_z(r_ref, c_ref, g_ref, bt_ref, s_ref, q_ref, parts):
    m = s_ref[0, 0:1, :] * (1.0 / N_NODES)
    msq = q_ref[0, 0:1, :] * (1.0 / N_NODES)
    inv_std = lax.rsqrt(jnp.maximum(msq - m * m, 0.0) + 1e-5)
    c = c_ref[:, 0:1]
    dinv = jnp.where(c > 0, 1.0 / c, 0.0)
    y = _load_raw(r_ref, parts) * dinv
    return jnp.maximum((y - m) * inv_std * g_ref[0] + bt_ref[0], 0.0)


def _bn_mm(raw, cnt, g, bt, s, q, wc, parts, x0c=None):
    nci, nco = wc.shape[0], wc.shape[1]
    has_res = x0c is not None

    def body(*refs):
        if has_res:
            r_ref, c_ref, g_ref, bt_ref, s_ref, q_ref, x0_ref, w_ref, o_ref = refs
        else:
            r_ref, c_ref, g_ref, bt_ref, s_ref, q_ref, w_ref, o_ref = refs
        kk = pl.program_id(2)
        z = _bn_z(r_ref, c_ref, g_ref, bt_ref, s_ref, q_ref, parts)
        if has_res:
            z = z + x0_ref[0]
        acc = jnp.dot(z, w_ref[0, 0], preferred_element_type=jnp.float32)[None]

        @pl.when(kk == 0)
        def _():
            o_ref[...] = acc

        @pl.when(kk != 0)
        def _():
            o_ref[...] += acc

    in_specs = [
        _raw_spec(parts, lambda o, r, kk: (kk, r, 0)),
        pl.BlockSpec((BN, C), lambda o, r, kk: (r, 0)),
        pl.BlockSpec((1, 1, C), lambda o, r, kk: (kk, 0, 0)),
        pl.BlockSpec((1, 1, C), lambda o, r, kk: (kk, 0, 0)),
        pl.BlockSpec((1, 8, C), lambda o, r, kk: (kk, 0, 0)),
        pl.BlockSpec((1, 8, C), lambda o, r, kk: (kk, 0, 0)),
    ]
    args = [raw, cnt, g.reshape(nci, 1, C), bt.reshape(nci, 1, C), s, q]
    if has_res:
        in_specs.append(pl.BlockSpec((1, BN, C), lambda o, r, kk: (kk, r, 0)))
        args.append(x0c)
    in_specs.append(pl.BlockSpec((1, 1, C, C), lambda o, r, kk: (kk, o, 0, 0)))
    args.append(wc)

    return pl.pallas_call(
        body,
        grid=(nco, N_NODES // BN, nci),
        in_specs=in_specs,
        out_specs=pl.BlockSpec((1, BN, C), lambda o, r, kk: (o, r, 0)),
        out_shape=jax.ShapeDtypeStruct((nco, N_NODES, C), jnp.float32),
    )(*args)


def _bn_final(raw, cnt, g, bt, s, q, parts):
    nc = raw.shape[1] if parts else raw.shape[0]

    def body(r_ref, c_ref, g_ref, bt_ref, s_ref, q_ref, o_ref):
        o_ref[...] = _bn_z(r_ref, c_ref, g_ref, bt_ref, s_ref, q_ref, parts)[None]

    return pl.pallas_call(
        body,
        grid=(nc, N_NODES // BN),
        in_specs=[
            _raw_spec(parts, lambda o, r: (o, r, 0)),
            pl.BlockSpec((BN, C), lambda o, r: (r, 0)),
            pl.BlockSpec((1, 1, C), lambda o, r: (o, 0, 0)),
            pl.BlockSpec((1, 1, C), lambda o, r: (o, 0, 0)),
            pl.BlockSpec((1, 8, C), lambda o, r: (o, 0, 0)),
            pl.BlockSpec((1, 8, C), lambda o, r: (o, 0, 0)),
        ],
        out_specs=pl.BlockSpec((1, BN, C), lambda o, r: (o, r, 0)),
        out_shape=jax.ShapeDtypeStruct((nc, N_NODES, C), jnp.float32),
    )(raw, cnt, g.reshape(nc, 1, C), bt.reshape(nc, 1, C), s, q)


def _chunk_w(w):
    di, do = w.shape
    nci, nco = di // C, do // C
    return w.reshape(nci, C, nco, C).transpose(0, 2, 1, 3)


def kernel(x, edge, W1, b1, g1, bt1, W2, b2, g2, bt2, W3, b3, g3, bt3,
           W4, b4, g4, bt4):
    nidx = edge[0]
    hidx = edge[1]

    nid_s = _tile_pad(nidx, SENT)
    hid_s = _tile_pad(hidx, SENT)
    nid_g = _make_gather_idx(_tile_pad(nidx, 0), 3)
    hid_g = _make_gather_idx(_tile_pad(hidx, 0), 3)
    nid_s2 = _tile_pad2(nidx, SENT)
    hid_s2 = _tile_pad2(hidx, SENT)
    nid_g2 = _make_gather_idx2(_tile_pad2(nidx, 0), 3)
    hid_g2 = _make_gather_idx2(_tile_pad2(hidx, 0), 3)

    ones = jnp.ones((128, C), jnp.float32)
    zeros = jnp.zeros((40, C), jnp.float32)

    sidx2 = jnp.stack([nid_s, hid_s])
    cnts = _sc_degrees()(sidx2, ones, zeros)
    d16 = cnts[0]
    bd16 = cnts[1]

    pass_full = _sc_pass(True)
    pass_half = _sc_pass(False)

    def conv6(xw):
        he_raw = pass_full(xw.reshape(6 * N_NODES, C), nid_g, hid_s, zeros)
        he_s = _scale_rows(he_raw, bd16, False)
        return pass_full(he_s.reshape(6 * N_NODES, C), hid_g, nid_s, zeros)

    def conv3(xw):
        he_raw = pass_half(xw.reshape(3 * N_NODES, C), nid_g2, hid_s2, zeros)
        he_s = _scale_rows(he_raw, bd16, True)
        return pass_half(he_s.reshape(3 * N_NODES, C), hid_g2, nid_s2, zeros)

    xw = _mm_in_flat(x, W1, 6)
    r1 = conv6(xw)
    s1, q1 = _stats(r1, d16, False)
    xw = _bn_mm(r1, d16, g1, bt1, s1, q1, _chunk_w(W2), False)
    r2 = conv6(xw)
    s2, q2 = _stats(r2, d16, False)
    xw = _bn_mm(r2, d16, g2, bt2, s2, q2, _chunk_w(W3), False)
    r3 = conv3(xw)
    s3, q3 = _stats(r3, d16, True)
    x0c = x.reshape(N_NODES, 3, C).transpose(1, 0, 2)
    xw = _bn_mm(r3, d16, g3, bt3, s3, q3, _chunk_w(W4), True, x0c=x0c)
    r4 = conv3(xw)
    s4, q4 = _stats(r4, d16, True)
    h = _bn_final(r4, d16, g4, bt4, s4, q4, True)
    return h.transpose(1, 0, 2).reshape(N_NODES, 3 * C)


# speedup vs baseline: 1.6773x; 1.0442x over previous
import functools

import jax
import jax.numpy as jnp
from jax import lax
from jax.experimental import pallas as pl
from jax.experimental.pallas import tpu as pltpu
from jax.experimental.pallas import tpu_sc as plsc

N_NODES = 10000
N_HE = 10000
N_EDGES = 160000
NT = 16
NCORE = 2
C = 128
EPT = N_EDGES // NT
ECH = 80
EPAD = ECH * 128 - EPT
EPT2 = N_EDGES // (NCORE * NT)
ECH2 = 40
EPAD2 = ECH2 * 128 - EPT2
ACC_ROWS = 10112
SENT = N_NODES
BN = 1000


def _tile_pad(idx, sentinel):
    a = idx.reshape(NT, EPT)
    a = jnp.pad(a, ((0, 0), (0, EPAD)), constant_values=sentinel)
    return a.reshape(NT, ECH, 128)


def _make_gather_idx(idx_pad, nch):
    nc = NCORE * nch
    shift = (jnp.arange(nc, dtype=jnp.int32) * N_NODES)[:, None, None, None]
    g = idx_pad[None] + shift
    g = g.reshape(nc, NT, 2, 40, 128)
    g = jnp.pad(g, ((0, 0), (0, 0), (0, 0), (0, 1), (0, 0)))
    g = g.reshape(NCORE, nch, NT, 2, 41, 128)
    return g.transpose(0, 2, 1, 3, 4, 5)


def _tile_pad2(idx, sentinel):
    a = idx.reshape(NCORE, NT, EPT2)
    a = jnp.pad(a, ((0, 0), (0, 0), (0, EPAD2)), constant_values=sentinel)
    return a.reshape(NCORE, NT, ECH2, 128)


def _make_gather_idx2(idx_pad2, nch):
    shift = (jnp.arange(nch, dtype=jnp.int32) * N_NODES).reshape(1, 1, nch, 1, 1)
    cshift = (jnp.arange(NCORE, dtype=jnp.int32) * (nch * N_NODES)
              ).reshape(NCORE, 1, 1, 1, 1)
    g = idx_pad2[:, :, None] + shift + cshift
    g = jnp.pad(g, ((0, 0), (0, 0), (0, 0), (0, 1), (0, 0)))
    return g[:, :, :, None]


def _sc_pass(full):
    nh = 2 if full else 1
    mesh = plsc.VectorSubcoreMesh(core_axis_name="c", subcore_axis_name="s")
    out_t = (jax.ShapeDtypeStruct((6, N_NODES, C), jnp.float32) if full
             else jax.ShapeDtypeStruct((NCORE, 3, N_NODES, C), jnp.float32))

    @functools.partial(
        pl.kernel, mesh=mesh,
        out_type=out_t,
        scratch_types=[
            pltpu.VMEM((41, 128), jnp.int32),
            pltpu.VMEM((40, 128), jnp.int32),
            pltpu.VMEM((128, C), jnp.float32),
            pltpu.VMEM((128, C), jnp.float32),
            pltpu.VMEM((40, C), jnp.float32),
            pltpu.VMEM_SHARED((ACC_ROWS, C), jnp.float32),
            pltpu.SemaphoreType.DMA,
            pltpu.SemaphoreType.DMA,
        ],
    )
    def k(src, gidx, sidx, zeros, out, gidx_v, sidx_v, rows_a, rows_b, zero_v,
          acc, sem_a, sem_b):
        core = lax.axis_index("c")
        s = lax.axis_index("s")
        pltpu.sync_copy(zeros, zero_v)
        if not full:
            pltpu.sync_copy(sidx.at[core, s], sidx_v)
        for cc in range(3):
            for z in range(15):
                pltpu.sync_copy(zero_v, acc.at[pl.ds(s * 632 + z * 40, 40)])
            pltpu.sync_copy(zero_v.at[pl.ds(0, 32)],
                            acc.at[pl.ds(s * 632 + 600, 32)])
            plsc.subcore_barrier()
            for h in range(nh):
                pltpu.sync_copy(gidx.at[core, s, cc, h], gidx_v)
                if full:
                    pltpu.sync_copy(sidx.at[s, pl.ds(h * 40, 40)], sidx_v)

                def body(j, carry):
                    pltpu.async_copy(src.at[gidx_v.at[j]], rows_a,
                                     sem_a).wait()
                    pltpu.sync_copy(rows_a, acc.at[sidx_v.at[j]], add=True)
                    return carry

                lax.fori_loop(0, 40, body, 0)
            plsc.subcore_barrier()
            if full:
                chunk = core * 3 + cc
                pltpu.sync_copy(acc.at[pl.ds(s * 624, 640)],
                                out.at[chunk, pl.ds(s * 624, 640)])
            else:
                pltpu.sync_copy(acc.at[pl.ds(s * 624, 640)],
                                out.at[core, cc, pl.ds(s * 624, 640)])
            plsc.subcore_barrier()

    return k


def _sc_degrees():
    mesh = plsc.VectorSubcoreMesh(core_axis_name="c", subcore_axis_name="s")

    @functools.partial(
        pl.kernel, mesh=mesh,
        out_type=jax.ShapeDtypeStruct((2, N_NODES, C), jnp.float32),
        scratch_types=[
            pltpu.VMEM((ECH, 128), jnp.int32),
            pltpu.VMEM((128, C), jnp.float32),
            pltpu.VMEM((40, C), jnp.float32),
            pltpu.VMEM_SHARED((ACC_ROWS, C), jnp.float32),
        ],
    )
    def k(sidx2, ones, zeros, out, sidx_v, ones_v, zero_v, acc):
        core = lax.axis_index("c")
        s = lax.axis_index("s")
        pltpu.sync_copy(sidx2.at[core, s], sidx_v)
        pltpu.sync_copy(ones, ones_v)
        pltpu.sync_copy(zeros, zero_v)
        for z in range(15):
            pltpu.sync_copy(zero_v, acc.at[pl.ds(s * 632 + z * 40, 40)])
        pltpu.sync_copy(zero_v.at[pl.ds(0, 32)],
                        acc.at[pl.ds(s * 632 + 600, 32)])
        plsc.subcore_barrier()

        def body(j, carry):
            pltpu.sync_copy(ones_v, acc.at[sidx_v.at[j]], add=True)
            return carry

        lax.fori_loop(0, ECH, body, 0)
        plsc.subcore_barrier()
        pltpu.sync_copy(acc.at[pl.ds(s * 624, 640)],
                        out.at[core, pl.ds(s * 624, 640)])

    return k


def _load_raw(r_ref, parts):
    return (r_ref[0, 0] + r_ref[1, 0]) if parts else r_ref[0]


def _raw_spec(parts, imap3):
    if parts:
        return pl.BlockSpec((2, 1, BN, C), lambda *g: (0,) + imap3(*g))
    return pl.BlockSpec((1, BN, C), imap3)


def _mm_in_flat(x, w, nco):
    din = x.shape[1]

    def body(x_ref, w_ref, o_ref):
        o_ref[...] = jnp.dot(x_ref[...], w_ref[...],
                             preferred_element_type=jnp.float32)[None]

    return pl.pallas_call(
        body,
        grid=(nco, N_NODES // BN),
        in_specs=[
            pl.BlockSpec((BN, din), lambda o, r: (r, 0)),
            pl.BlockSpec((din, C), lambda o, r: (0, o)),
        ],
        out_specs=pl.BlockSpec((1, BN, C), lambda o, r: (o, r, 0)),
        out_shape=jax.ShapeDtypeStruct((nco, N_NODES, C), jnp.float32),
    )(x, w)


def _scale_rows(raw, cnt, parts):
    nc = raw.shape[1] if parts else raw.shape[0]

    def body(r_ref, c_ref, o_ref):
        c = c_ref[:, 0:1]
        inv = jnp.where(c > 0, 1.0 / c, 0.0)
        o_ref[...] = (_load_raw(r_ref, parts) * inv)[None]

    return pl.pallas_call(
        body,
        grid=(nc, N_NODES // BN),
        in_specs=[
            _raw_spec(parts, lambda o, r: (o, r, 0)),
            pl.BlockSpec((BN, C), lambda o, r: (r, 0)),
        ],
        out_specs=pl.BlockSpec((1, BN, C), lambda o, r: (o, r, 0)),
        out_shape=jax.ShapeDtypeStruct((nc, N_NODES, C), jnp.float32),
    )(raw, cnt)


def _stats(raw, cnt, parts):
    nc = raw.shape[1] if parts else raw.shape[0]

    def body(r_ref, c_ref, s_ref, q_ref):
        r = pl.program_id(1)
        c = c_ref[:, 0:1]
        inv = jnp.where(c > 0, 1.0 / c, 0.0)
        y = _load_raw(r_ref, parts) * inv
        s1 = jnp.broadcast_to(jnp.sum(y, axis=0, keepdims=True), (8, C))[None]
        q1 = jnp.broadcast_to(jnp.sum(y * y, axis=0, keepdims=True), (8, C))[None]

        @pl.when(r == 0)
        def _():
            s_ref[...] = s1
            q_ref[...] = q1

        @pl.when(r != 0)
        def _():
            s_ref[...] += s1
            q_ref[...] += q1

    return pl.pallas_call(
        body,
        grid=(nc, N_NODES // BN),
        in_specs=[
            _raw_spec(parts, lambda o, r: (o, r, 0)),
            pl.BlockSpec((BN, C), lambda o, r: (r, 0)),
        ],
        out_specs=[
            pl.BlockSpec((1, 8, C), lambda o, r: (o, 0, 0)),
            pl.BlockSpec((1, 8, C), lambda o, r: (o, 0, 0)),
        ],
        out_shape=[
            jax.ShapeDtypeStruct((nc, 8, C), jnp.float32),
            jax.ShapeDtypeStruct((nc, 8, C), jnp.float32),
        ],
    )(raw, cnt)


def _bn_z(r_ref, c_ref, g_ref, bt_ref, s_ref, q_ref, parts):
    m = s_ref[0, 0:1, :] * (1.0 / N_NODES)
    msq = q_ref[0, 0:1, :] * (1.0 / N_NODES)
    inv_std = lax.rsqrt(jnp.maximum(msq - m * m, 0.0) + 1e-5)
    c = c_ref[:, 0:1]
    dinv = jnp.where(c > 0, 1.0 / c, 0.0)
    y = _load_raw(r_ref, parts) * dinv
    return jnp.maximum((y - m) * inv_std * g_ref[0] + bt_ref[0], 0.0)


def _bn_mm(raw, cnt, g, bt, s, q, wc, parts, x0c=None):
    nci, nco = wc.shape[0], wc.shape[1]
    has_res = x0c is not None

    def body(*refs):
        if has_res:
            r_ref, c_ref, g_ref, bt_ref, s_ref, q_ref, x0_ref, w_ref, o_ref = refs
        else:
            r_ref, c_ref, g_ref, bt_ref, s_ref, q_ref, w_ref, o_ref = refs
        kk = pl.program_id(2)
        z = _bn_z(r_ref, c_ref, g_ref, bt_ref, s_ref, q_ref, parts)
        if has_res:
            z = z + x0_ref[0]
        acc = jnp.dot(z, w_ref[0, 0], preferred_element_type=jnp.float32)[None]

        @pl.when(kk == 0)
        def _():
            o_ref[...] = acc

        @pl.when(kk != 0)
        def _():
            o_ref[...] += acc

    in_specs = [
        _raw_spec(parts, lambda o, r, kk: (kk, r, 0)),
        pl.BlockSpec((BN, C), lambda o, r, kk: (r, 0)),
        pl.BlockSpec((1, 1, C), lambda o, r, kk: (kk, 0, 0)),
        pl.BlockSpec((1, 1, C), lambda o, r, kk: (kk, 0, 0)),
        pl.BlockSpec((1, 8, C), lambda o, r, kk: (kk, 0, 0)),
        pl.BlockSpec((1, 8, C), lambda o, r, kk: (kk, 0, 0)),
    ]
    args = [raw, cnt, g.reshape(nci, 1, C), bt.reshape(nci, 1, C), s, q]
    if has_res:
        in_specs.append(pl.BlockSpec((1, BN, C), lambda o, r, kk: (kk, r, 0)))
        args.append(x0c)
    in_specs.append(pl.BlockSpec((1, 1, C, C), lambda o, r, kk: (kk, o, 0, 0)))
    args.append(wc)

    return pl.pallas_call(
        body,
        grid=(nco, N_NODES // BN, nci),
        in_specs=in_specs,
        out_specs=pl.BlockSpec((1, BN, C), lambda o, r, kk: (o, r, 0)),
        out_shape=jax.ShapeDtypeStruct((nco, N_NODES, C), jnp.float32),
    )(*args)


def _bn_final(raw, cnt, g, bt, s, q, parts):
    nc = raw.shape[1] if parts else raw.shape[0]

    def body(r_ref, c_ref, g_ref, bt_ref, s_ref, q_ref, o_ref):
        o_ref[...] = _bn_z(r_ref, c_ref, g_ref, bt_ref, s_ref, q_ref, parts)[None]

    return pl.pallas_call(
        body,
        grid=(nc, N_NODES // BN),
        in_specs=[
            _raw_spec(parts, lambda o, r: (o, r, 0)),
            pl.BlockSpec((BN, C), lambda o, r: (r, 0)),
            pl.BlockSpec((1, 1, C), lambda o, r: (o, 0, 0)),
            pl.BlockSpec((1, 1, C), lambda o, r: (o, 0, 0)),
            pl.BlockSpec((1, 8, C), lambda o, r: (o, 0, 0)),
            pl.BlockSpec((1, 8, C), lambda o, r: (o, 0, 0)),
        ],
        out_specs=pl.BlockSpec((1, BN, C), lambda o, r: (o, r, 0)),
        out_shape=jax.ShapeDtypeStruct((nc, N_NODES, C), jnp.float32),
    )(raw, cnt, g.reshape(nc, 1, C), bt.reshape(nc, 1, C), s, q)


def _chunk_w(w):
    di, do = w.shape
    nci, nco = di // C, do // C
    return w.reshape(nci, C, nco, C).transpose(0, 2, 1, 3)


def kernel(x, edge, W1, b1, g1, bt1, W2, b2, g2, bt2, W3, b3, g3, bt3,
           W4, b4, g4, bt4):
    nidx = edge[0]
    hidx = edge[1]

    nid_s = _tile_pad(nidx, SENT)
    hid_s = _tile_pad(hidx, SENT)
    nid_g = _make_gather_idx(_tile_pad(nidx, 0), 3)
    hid_g = _make_gather_idx(_tile_pad(hidx, 0), 3)
    nid_s2 = _tile_pad2(nidx, SENT)
    hid_s2 = _tile_pad2(hidx, SENT)
    nid_g2 = _make_gather_idx2(_tile_pad2(nidx, 0), 3)
    hid_g2 = _make_gather_idx2(_tile_pad2(hidx, 0), 3)

    ones = jnp.ones((128, C), jnp.float32)
    zeros = jnp.zeros((40, C), jnp.float32)

    sidx2 = jnp.stack([nid_s, hid_s])
    cnts = _sc_degrees()(sidx2, ones, zeros)
    d16 = cnts[0]
    bd16 = cnts[1]

    pass_full = _sc_pass(True)
    pass_half = _sc_pass(False)

    def conv6(xw):
        he_raw = pass_full(xw.reshape(6 * N_NODES, C), nid_g, hid_s, zeros)
        he_s = _scale_rows(he_raw, bd16, False)
        return pass_full(he_s.reshape(6 * N_NODES, C), hid_g, nid_s, zeros)

    def conv3(xw):
        x2 = jnp.concatenate([xw.reshape(3 * N_NODES, C)] * 2)
        he_raw = pass_half(x2, nid_g2, hid_s2, zeros)
        he_s = _scale_rows(he_raw, bd16, True)
        h2 = jnp.concatenate([he_s.reshape(3 * N_NODES, C)] * 2)
        return pass_half(h2, hid_g2, nid_s2, zeros)

    xw = _mm_in_flat(x, W1, 6)
    r1 = conv6(xw)
    s1, q1 = _stats(r1, d16, False)
    xw = _bn_mm(r1, d16, g1, bt1, s1, q1, _chunk_w(W2), False)
    r2 = conv6(xw)
    s2, q2 = _stats(r2, d16, False)
    xw = _bn_mm(r2, d16, g2, bt2, s2, q2, _chunk_w(W3), False)
    r3 = conv3(xw)
    s3, q3 = _stats(r3, d16, True)
    x0c = x.reshape(N_NODES, 3, C).transpose(1, 0, 2)
    xw = _bn_mm(r3, d16, g3, bt3, s3, q3, _chunk_w(W4), True, x0c=x0c)
    r4 = conv3(xw)
    s4, q4 = _stats(r4, d16, True)
    h = _bn_final(r4, d16, g4, bt4, s4, q4, True)
    return h.transpose(1, 0, 2).reshape(N_NODES, 3 * C)


# speedup vs baseline: 1.7446x; 1.0401x over previous
import functools

import jax
import jax.numpy as jnp
from jax import lax
from jax.experimental import pallas as pl
from jax.experimental.pallas import tpu as pltpu
from jax.experimental.pallas import tpu_sc as plsc

N_NODES = 10000
N_HE = 10000
N_EDGES = 160000
NT = 16
NCORE = 2
C = 128
EPT = N_EDGES // NT
ECH = 80
EPAD = ECH * 128 - EPT
EPT2 = N_EDGES // (NCORE * NT)
ECH2 = 40
EPAD2 = ECH2 * 128 - EPT2
ACC_ROWS = 10112
SENT = N_NODES
BN = 1000


def _tile_pad(idx, sentinel):
    a = idx.reshape(NT, EPT)
    a = jnp.pad(a, ((0, 0), (0, EPAD)), constant_values=sentinel)
    return a.reshape(NT, ECH, 128)


def _make_gather_idx(idx_pad, nch):
    nc = NCORE * nch
    shift = (jnp.arange(nc, dtype=jnp.int32) * N_NODES)[:, None, None, None]
    g = idx_pad[None] + shift
    g = g.reshape(nc, NT, 2, 40, 128)
    g = jnp.pad(g, ((0, 0), (0, 0), (0, 0), (0, 1), (0, 0)))
    g = g.reshape(NCORE, nch, NT, 2, 41, 128)
    return g.transpose(0, 2, 1, 3, 4, 5)


def _tile_pad2(idx, sentinel):
    a = idx.reshape(NCORE, NT, EPT2)
    a = jnp.pad(a, ((0, 0), (0, 0), (0, EPAD2)), constant_values=sentinel)
    return a.reshape(NCORE, NT, ECH2, 128)


def _make_gather_idx2(idx_pad2, nch):
    shift = (jnp.arange(nch, dtype=jnp.int32) * N_NODES).reshape(1, 1, nch, 1, 1)
    cshift = (jnp.arange(NCORE, dtype=jnp.int32) * (nch * N_NODES)
              ).reshape(NCORE, 1, 1, 1, 1)
    g = idx_pad2[:, :, None] + shift + cshift
    g = jnp.pad(g, ((0, 0), (0, 0), (0, 0), (0, 1), (0, 0)))
    return g[:, :, :, None]


def _sc_pass(full):
    nh = 2 if full else 1
    mesh = plsc.VectorSubcoreMesh(core_axis_name="c", subcore_axis_name="s")
    out_t = (jax.ShapeDtypeStruct((6, N_NODES, C), jnp.float32) if full
             else jax.ShapeDtypeStruct((NCORE, 3, N_NODES, C), jnp.float32))

    @functools.partial(
        pl.kernel, mesh=mesh,
        out_type=out_t,
        scratch_types=[
            pltpu.VMEM((41, 128), jnp.int32),
            pltpu.VMEM((nh * 40, 128), jnp.int32),
            pltpu.VMEM((128, C), jnp.float32),
            pltpu.VMEM((128, C), jnp.float32),
            pltpu.VMEM_SHARED((ACC_ROWS, C), jnp.float32),
            pltpu.SemaphoreType.DMA,
            pltpu.SemaphoreType.DMA,
            pltpu.SemaphoreType.DMA,
        ],
    )
    def k(src, gidx, sidx, zeros, out, gidx_v, sidx_v, rows_a, rows_b,
          acc, sem_a, sem_b, sem_s):
        core = lax.axis_index("c")
        s = lax.axis_index("s")
        if full:
            pltpu.sync_copy(sidx.at[s], sidx_v)
        else:
            pltpu.sync_copy(sidx.at[core, s], sidx_v)
        for cc in range(3):
            pltpu.sync_copy(zeros, acc.at[pl.ds(s * 632, 632)])
            plsc.subcore_barrier()
            for h in range(nh):
                pltpu.sync_copy(gidx.at[core, s, cc, h], gidx_v)

                def pair(i, carry):
                    ja = 2 * i
                    pltpu.async_copy(src.at[gidx_v.at[ja]], rows_a,
                                     sem_a).wait()
                    cps = pltpu.async_copy(
                        rows_a, acc.at[sidx_v.at[h * 40 + ja]], sem_s,
                        add=True)
                    pltpu.async_copy(src.at[gidx_v.at[ja + 1]], rows_b,
                                     sem_b).wait()
                    cps.wait()
                    pltpu.sync_copy(rows_b, acc.at[sidx_v.at[h * 40 + ja + 1]],
                                    add=True)
                    return carry

                lax.fori_loop(0, 20, pair, 0)
            plsc.subcore_barrier()
            if full:
                chunk = core * 3 + cc
                pltpu.sync_copy(acc.at[pl.ds(s * 624, 640)],
                                out.at[chunk, pl.ds(s * 624, 640)])
            else:
                pltpu.sync_copy(acc.at[pl.ds(s * 624, 640)],
                                out.at[core, cc, pl.ds(s * 624, 640)])
            plsc.subcore_barrier()

    return k


def _sc_degrees():
    mesh = plsc.VectorSubcoreMesh(core_axis_name="c", subcore_axis_name="s")

    @functools.partial(
        pl.kernel, mesh=mesh,
        out_type=jax.ShapeDtypeStruct((2, N_NODES, C), jnp.float32),
        scratch_types=[
            pltpu.VMEM((ECH, 128), jnp.int32),
            pltpu.VMEM((128, C), jnp.float32),
            pltpu.VMEM_SHARED((ACC_ROWS, C), jnp.float32),
        ],
    )
    def k(sidx2, ones, zeros, out, sidx_v, ones_v, acc):
        core = lax.axis_index("c")
        s = lax.axis_index("s")
        pltpu.sync_copy(sidx2.at[core, s], sidx_v)
        pltpu.sync_copy(ones, ones_v)
        pltpu.sync_copy(zeros, acc.at[pl.ds(s * 632, 632)])
        plsc.subcore_barrier()

        def body(j, carry):
            pltpu.sync_copy(ones_v, acc.at[sidx_v.at[j]], add=True)
            return carry

        lax.fori_loop(0, ECH, body, 0)
        plsc.subcore_barrier()
        pltpu.sync_copy(acc.at[pl.ds(s * 624, 640)],
                        out.at[core, pl.ds(s * 624, 640)])

    return k


def _load_raw(r_ref, parts):
    return (r_ref[0, 0] + r_ref[1, 0]) if parts else r_ref[0]


def _raw_spec(parts, imap3):
    if parts:
        return pl.BlockSpec((2, 1, BN, C), lambda *g: (0,) + imap3(*g))
    return pl.BlockSpec((1, BN, C), imap3)


def _mm_in_flat(x, w, nco):
    din = x.shape[1]

    def body(x_ref, w_ref, o_ref):
        o_ref[...] = jnp.dot(x_ref[...], w_ref[...],
                             preferred_element_type=jnp.float32)[None]

    return pl.pallas_call(
        body,
        grid=(nco, N_NODES // BN),
        in_specs=[
            pl.BlockSpec((BN, din), lambda o, r: (r, 0)),
            pl.BlockSpec((din, C), lambda o, r: (0, o)),
        ],
        out_specs=pl.BlockSpec((1, BN, C), lambda o, r: (o, r, 0)),
        out_shape=jax.ShapeDtypeStruct((nco, N_NODES, C), jnp.float32),
    )(x, w)


def _scale_rows(raw, cnt, parts):
    nc = raw.shape[1] if parts else raw.shape[0]

    def body(r_ref, c_ref, o_ref):
        c = c_ref[:, 0:1]
        inv = jnp.where(c > 0, 1.0 / c, 0.0)
        o_ref[...] = (_load_raw(r_ref, parts) * inv)[None]

    return pl.pallas_call(
        body,
        grid=(nc, N_NODES // BN),
        in_specs=[
            _raw_spec(parts, lambda o, r: (o, r, 0)),
            pl.BlockSpec((BN, C), lambda o, r: (r, 0)),
        ],
        out_specs=pl.BlockSpec((1, BN, C), lambda o, r: (o, r, 0)),
        out_shape=jax.ShapeDtypeStruct((nc, N_NODES, C), jnp.float32),
    )(raw, cnt)


def _stats(raw, cnt, parts):
    nc = raw.shape[1] if parts else raw.shape[0]

    def body(r_ref, c_ref, s_ref, q_ref):
        r = pl.program_id(1)
        c = c_ref[:, 0:1]
        inv = jnp.where(c > 0, 1.0 / c, 0.0)
        y = _load_raw(r_ref, parts) * inv
        s1 = jnp.broadcast_to(jnp.sum(y, axis=0, keepdims=True), (8, C))[None]
        q1 = jnp.broadcast_to(jnp.sum(y * y, axis=0, keepdims=True), (8, C))[None]

        @pl.when(r == 0)
        def _():
            s_ref[...] = s1
            q_ref[...] = q1

        @pl.when(r != 0)
        def _():
            s_ref[...] += s1
            q_ref[...] += q1

    return pl.pallas_call(
        body,
        grid=(nc, N_NODES // BN),
        in_specs=[
            _raw_spec(parts, lambda o, r: (o, r, 0)),
            pl.BlockSpec((BN, C), lambda o, r: (r, 0)),
        ],
        out_specs=[
            pl.BlockSpec((1, 8, C), lambda o, r: (o, 0, 0)),
            pl.BlockSpec((1, 8, C), lambda o, r: (o, 0, 0)),
        ],
        out_shape=[
            jax.ShapeDtypeStruct((nc, 8, C), jnp.float32),
            jax.ShapeDtypeStruct((nc, 8, C), jnp.float32),
        ],
    )(raw, cnt)


def _bn_z(r_ref, c_ref, g_ref, bt_ref, s_ref, q_ref, parts):
    m = s_ref[0, 0:1, :] * (1.0 / N_NODES)
    msq = q_ref[0, 0:1, :] * (1.0 / N_NODES)
    inv_std = lax.rsqrt(jnp.maximum(msq - m * m, 0.0) + 1e-5)
    c = c_ref[:, 0:1]
    dinv = jnp.where(c > 0, 1.0 / c, 0.0)
    y = _load_raw(r_ref, parts) * dinv
    return jnp.maximum((y - m) * inv_std * g_ref[0] + bt_ref[0], 0.0)


def _bn_mm(raw, cnt, g, bt, s, q, wc, parts, x0c=None):
    nci, nco = wc.shape[0], wc.shape[1]
    has_res = x0c is not None

    def body(*refs):
        if has_res:
            r_ref, c_ref, g_ref, bt_ref, s_ref, q_ref, x0_ref, w_ref, o_ref = refs
        else:
            r_ref, c_ref, g_ref, bt_ref, s_ref, q_ref, w_ref, o_ref = refs
        kk = pl.program_id(2)
        z = _bn_z(r_ref, c_ref, g_ref, bt_ref, s_ref, q_ref, parts)
        if has_res:
            z = z + x0_ref[0]
        acc = jnp.dot(z, w_ref[0, 0], preferred_element_type=jnp.float32)[None]

        @pl.when(kk == 0)
        def _():
            o_ref[...] = acc

        @pl.when(kk != 0)
        def _():
            o_ref[...] += acc

    in_specs = [
        _raw_spec(parts, lambda o, r, kk: (kk, r, 0)),
        pl.BlockSpec((BN, C), lambda o, r, kk: (r, 0)),
        pl.BlockSpec((1, 1, C), lambda o, r, kk: (kk, 0, 0)),
        pl.BlockSpec((1, 1, C), lambda o, r, kk: (kk, 0, 0)),
        pl.BlockSpec((1, 8, C), lambda o, r, kk: (kk, 0, 0)),
        pl.BlockSpec((1, 8, C), lambda o, r, kk: (kk, 0, 0)),
    ]
    args = [raw, cnt, g.reshape(nci, 1, C), bt.reshape(nci, 1, C), s, q]
    if has_res:
        in_specs.append(pl.BlockSpec((1, BN, C), lambda o, r, kk: (kk, r, 0)))
        args.append(x0c)
    in_specs.append(pl.BlockSpec((1, 1, C, C), lambda o, r, kk: (kk, o, 0, 0)))
    args.append(wc)

    return pl.pallas_call(
        body,
        grid=(nco, N_NODES // BN, nci),
        in_specs=in_specs,
        out_specs=pl.BlockSpec((1, BN, C), lambda o, r, kk: (o, r, 0)),
        out_shape=jax.ShapeDtypeStruct((nco, N_NODES, C), jnp.float32),
    )(*args)


def _bn_final(raw, cnt, g, bt, s, q, parts):
    nc = raw.shape[1] if parts else raw.shape[0]

    def body(r_ref, c_ref, g_ref, bt_ref, s_ref, q_ref, o_ref):
        o_ref[...] = _bn_z(r_ref, c_ref, g_ref, bt_ref, s_ref, q_ref, parts)[None]

    return pl.pallas_call(
        body,
        grid=(nc, N_NODES // BN),
        in_specs=[
            _raw_spec(parts, lambda o, r: (o, r, 0)),
            pl.BlockSpec((BN, C), lambda o, r: (r, 0)),
            pl.BlockSpec((1, 1, C), lambda o, r: (o, 0, 0)),
            pl.BlockSpec((1, 1, C), lambda o, r: (o, 0, 0)),
            pl.BlockSpec((1, 8, C), lambda o, r: (o, 0, 0)),
            pl.BlockSpec((1, 8, C), lambda o, r: (o, 0, 0)),
        ],
        out_specs=pl.BlockSpec((1, BN, C), lambda o, r: (o, r, 0)),
        out_shape=jax.ShapeDtypeStruct((nc, N_NODES, C), jnp.float32),
    )(raw, cnt, g.reshape(nc, 1, C), bt.reshape(nc, 1, C), s, q)


def _chunk_w(w):
    di, do = w.shape
    nci, nco = di // C, do // C
    return w.reshape(nci, C, nco, C).transpose(0, 2, 1, 3)


def kernel(x, edge, W1, b1, g1, bt1, W2, b2, g2, bt2, W3, b3, g3, bt3,
           W4, b4, g4, bt4):
    nidx = edge[0]
    hidx = edge[1]

    nid_s = _tile_pad(nidx, SENT)
    hid_s = _tile_pad(hidx, SENT)
    nid_g = _make_gather_idx(_tile_pad(nidx, 0), 3)
    hid_g = _make_gather_idx(_tile_pad(hidx, 0), 3)
    nid_s2 = _tile_pad2(nidx, SENT)
    hid_s2 = _tile_pad2(hidx, SENT)
    nid_g2 = _make_gather_idx2(_tile_pad2(nidx, 0), 3)
    hid_g2 = _make_gather_idx2(_tile_pad2(hidx, 0), 3)

    ones = jnp.ones((128, C), jnp.float32)
    zeros = jnp.zeros((632, C), jnp.float32)

    sidx2 = jnp.stack([nid_s, hid_s])
    cnts = _sc_degrees()(sidx2, ones, zeros)
    d16 = cnts[0]
    bd16 = cnts[1]

    pass_full = _sc_pass(True)
    pass_half = _sc_pass(False)

    def conv6(xw):
        he_raw = pass_full(xw.reshape(6 * N_NODES, C), nid_g, hid_s, zeros)
        he_s = _scale_rows(he_raw, bd16, False)
        return pass_full(he_s.reshape(6 * N_NODES, C), hid_g, nid_s, zeros)

    def conv3(xw):
        x2 = jnp.concatenate([xw.reshape(3 * N_NODES, C)] * 2)
        he_raw = pass_half(x2, nid_g2, hid_s2, zeros)
        he_s = _scale_rows(he_raw, bd16, True)
        h2 = jnp.concatenate([he_s.reshape(3 * N_NODES, C)] * 2)
        return pass_half(h2, hid_g2, nid_s2, zeros)

    xw = _mm_in_flat(x, W1, 6)
    r1 = conv6(xw)
    s1, q1 = _stats(r1, d16, False)
    xw = _bn_mm(r1, d16, g1, bt1, s1, q1, _chunk_w(W2), False)
    r2 = conv6(xw)
    s2, q2 = _stats(r2, d16, False)
    xw = _bn_mm(r2, d16, g2, bt2, s2, q2, _chunk_w(W3), False)
    r3 = conv3(xw)
    s3, q3 = _stats(r3, d16, True)
    x0c = x.reshape(N_NODES, 3, C).transpose(1, 0, 2)
    xw = _bn_mm(r3, d16, g3, bt3, s3, q3, _chunk_w(W4), True, x0c=x0c)
    r4 = conv3(xw)
    s4, q4 = _stats(r4, d16, True)
    h = _bn_final(r4, d16, g4, bt4, s4, q4, True)
    return h.transpose(1, 0, 2).reshape(N_NODES, 3 * C)


# speedup vs baseline: 1.8479x; 1.0592x over previous
import functools

import jax
import jax.numpy as jnp
from jax import lax
from jax.experimental import pallas as pl
from jax.experimental.pallas import tpu as pltpu
from jax.experimental.pallas import tpu_sc as plsc

N_NODES = 10000
N_HE = 10000
N_EDGES = 160000
NT = 16
NCORE = 2
C = 128
EPT = N_EDGES // NT
ECH = 80
EPAD = ECH * 128 - EPT
EPT2 = N_EDGES // (NCORE * NT)
ECH2 = 40
EPAD2 = ECH2 * 128 - EPT2
ACC_ROWS = 10112
SENT = N_NODES
BN = 1000


def _tile_pad(idx, sentinel):
    a = idx.reshape(NT, EPT)
    a = jnp.pad(a, ((0, 0), (0, EPAD)), constant_values=sentinel)
    return a.reshape(NT, ECH, 128)


def _make_gather_idx(idx_pad, nch):
    nc = NCORE * nch
    shift = (jnp.arange(nc, dtype=jnp.int32) * N_NODES)[:, None, None, None]
    g = idx_pad[None] + shift
    g = g.reshape(nc, NT, 2, 40, 128)
    g = jnp.pad(g, ((0, 0), (0, 0), (0, 0), (0, 1), (0, 0)))
    g = g.reshape(NCORE, nch, NT, 2, 41, 128)
    return g.transpose(0, 2, 1, 3, 4, 5)


def _tile_pad2(idx, sentinel):
    a = idx.reshape(NCORE, NT, EPT2)
    a = jnp.pad(a, ((0, 0), (0, 0), (0, EPAD2)), constant_values=sentinel)
    return a.reshape(NCORE, NT, ECH2, 128)


def _make_gather_idx2(idx_pad2, nch):
    shift = (jnp.arange(nch, dtype=jnp.int32) * N_NODES).reshape(1, 1, nch, 1, 1)
    cshift = (jnp.arange(NCORE, dtype=jnp.int32) * (nch * N_NODES)
              ).reshape(NCORE, 1, 1, 1, 1)
    g = idx_pad2[:, :, None] + shift + cshift
    g = jnp.pad(g, ((0, 0), (0, 0), (0, 0), (0, 1), (0, 0)))
    return g[:, :, :, None]


def _sc_pass(full):
    nh = 2 if full else 1
    mesh = plsc.VectorSubcoreMesh(core_axis_name="c", subcore_axis_name="s")
    out_t = (jax.ShapeDtypeStruct((6, N_NODES, C), jnp.float32) if full
             else jax.ShapeDtypeStruct((NCORE, 3, N_NODES, C), jnp.float32))

    @functools.partial(
        pl.kernel, mesh=mesh,
        out_type=out_t,
        scratch_types=[
            pltpu.VMEM((41, 128), jnp.int32),
            pltpu.VMEM((nh * 40, 128), jnp.int32),
            pltpu.VMEM((128, C), jnp.float32),
            pltpu.VMEM((128, C), jnp.float32),
            pltpu.VMEM_SHARED((ACC_ROWS, C), jnp.float32),
            pltpu.SemaphoreType.DMA,
            pltpu.SemaphoreType.DMA,
            pltpu.SemaphoreType.DMA,
        ],
    )
    def k(src, gidx, sidx, zeros, out, gidx_v, sidx_v, rows_a, rows_b,
          acc, sem_a, sem_b, sem_s):
        core = lax.axis_index("c")
        s = lax.axis_index("s")
        if full:
            pltpu.sync_copy(sidx.at[s], sidx_v)
        else:
            pltpu.sync_copy(sidx.at[core, s], sidx_v)
        for cc in range(3):
            pltpu.sync_copy(zeros, acc.at[pl.ds(s * 632, 632)])
            plsc.subcore_barrier()
            for h in range(nh):
                pltpu.sync_copy(gidx.at[core, s, cc, h], gidx_v)

                def drain():
                    pltpu.make_async_copy(rows_a, acc.at[sidx_v.at[0]],
                                          sem_s).wait()

                def pair(i, carry):
                    ja = 2 * i

                    @pl.when(i > 0)
                    def _():
                        drain()

                    pltpu.async_copy(src.at[gidx_v.at[ja]], rows_a,
                                     sem_a).wait()
                    pltpu.async_copy(rows_a, acc.at[sidx_v.at[h * 40 + ja]],
                                     sem_s, add=True)

                    @pl.when(i > 0)
                    def _():
                        drain()

                    pltpu.async_copy(src.at[gidx_v.at[ja + 1]], rows_b,
                                     sem_b).wait()
                    pltpu.async_copy(rows_b,
                                     acc.at[sidx_v.at[h * 40 + ja + 1]],
                                     sem_s, add=True)
                    return carry

                lax.fori_loop(0, 20, pair, 0)
                drain()
                drain()
            plsc.subcore_barrier()
            if full:
                chunk = core * 3 + cc
                pltpu.sync_copy(acc.at[pl.ds(s * 624, 640)],
                                out.at[chunk, pl.ds(s * 624, 640)])
            else:
                pltpu.sync_copy(acc.at[pl.ds(s * 624, 640)],
                                out.at[core, cc, pl.ds(s * 624, 640)])
            plsc.subcore_barrier()

    return k


def _sc_degrees():
    mesh = plsc.VectorSubcoreMesh(core_axis_name="c", subcore_axis_name="s")

    @functools.partial(
        pl.kernel, mesh=mesh,
        out_type=jax.ShapeDtypeStruct((2, N_NODES, C), jnp.float32),
        scratch_types=[
            pltpu.VMEM((ECH, 128), jnp.int32),
            pltpu.VMEM((128, C), jnp.float32),
            pltpu.VMEM_SHARED((ACC_ROWS, C), jnp.float32),
        ],
    )
    def k(sidx2, ones, zeros, out, sidx_v, ones_v, acc):
        core = lax.axis_index("c")
        s = lax.axis_index("s")
        pltpu.sync_copy(sidx2.at[core, s], sidx_v)
        pltpu.sync_copy(ones, ones_v)
        pltpu.sync_copy(zeros, acc.at[pl.ds(s * 632, 632)])
        plsc.subcore_barrier()

        def body(j, carry):
            pltpu.sync_copy(ones_v, acc.at[sidx_v.at[j]], add=True)
            return carry

        lax.fori_loop(0, ECH, body, 0)
        plsc.subcore_barrier()
        pltpu.sync_copy(acc.at[pl.ds(s * 624, 640)],
                        out.at[core, pl.ds(s * 624, 640)])

    return k


def _load_raw(r_ref, parts):
    return (r_ref[0, 0] + r_ref[1, 0]) if parts else r_ref[0]


def _raw_spec(parts, imap3):
    if parts:
        return pl.BlockSpec((2, 1, BN, C), lambda *g: (0,) + imap3(*g))
    return pl.BlockSpec((1, BN, C), imap3)


def _mm_in_flat(x, w, nco):
    din = x.shape[1]

    def body(x_ref, w_ref, o_ref):
        o_ref[...] = jnp.dot(x_ref[...], w_ref[...],
                             preferred_element_type=jnp.float32)[None]

    return pl.pallas_call(
        body,
        grid=(nco, N_NODES // BN),
        in_specs=[
            pl.BlockSpec((BN, din), lambda o, r: (r, 0)),
            pl.BlockSpec((din, C), lambda o, r: (0, o)),
        ],
        out_specs=pl.BlockSpec((1, BN, C), lambda o, r: (o, r, 0)),
        out_shape=jax.ShapeDtypeStruct((nco, N_NODES, C), jnp.float32),
    )(x, w)


def _scale_rows(raw, cnt, parts):
    nc = raw.shape[1] if parts else raw.shape[0]

    def body(r_ref, c_ref, o_ref):
        c = c_ref[:, 0:1]
        inv = jnp.where(c > 0, 1.0 / c, 0.0)
        o_ref[...] = (_load_raw(r_ref, parts) * inv)[None]

    return pl.pallas_call(
        body,
        grid=(nc, N_NODES // BN),
        in_specs=[
            _raw_spec(parts, lambda o, r: (o, r, 0)),
            pl.BlockSpec((BN, C), lambda o, r: (r, 0)),
        ],
        out_specs=pl.BlockSpec((1, BN, C), lambda o, r: (o, r, 0)),
        out_shape=jax.ShapeDtypeStruct((nc, N_NODES, C), jnp.float32),
    )(raw, cnt)


def _stats(raw, cnt, parts):
    nc = raw.shape[1] if parts else raw.shape[0]

    def body(r_ref, c_ref, s_ref, q_ref):
        r = pl.program_id(1)
        c = c_ref[:, 0:1]
        inv = jnp.where(c > 0, 1.0 / c, 0.0)
        y = _load_raw(r_ref, parts) * inv
        s1 = jnp.broadcast_to(jnp.sum(y, axis=0, keepdims=True), (8, C))[None]
        q1 = jnp.broadcast_to(jnp.sum(y * y, axis=0, keepdims=True), (8, C))[None]

        @pl.when(r == 0)
        def _():
            s_ref[...] = s1
            q_ref[...] = q1

        @pl.when(r != 0)
        def _():
            s_ref[...] += s1
            q_ref[...] += q1

    return pl.pallas_call(
        body,
        grid=(nc, N_NODES // BN),
        in_specs=[
            _raw_spec(parts, lambda o, r: (o, r, 0)),
            pl.BlockSpec((BN, C), lambda o, r: (r, 0)),
        ],
        out_specs=[
            pl.BlockSpec((1, 8, C), lambda o, r: (o, 0, 0)),
            pl.BlockSpec((1, 8, C), lambda o, r: (o, 0, 0)),
        ],
        out_shape=[
            jax.ShapeDtypeStruct((nc, 8, C), jnp.float32),
            jax.ShapeDtypeStruct((nc, 8, C), jnp.float32),
        ],
    )(raw, cnt)


def _bn_z(r_ref, c_ref, g_ref, bt_ref, s_ref, q_ref, parts):
    m = s_ref[0, 0:1, :] * (1.0 / N_NODES)
    msq = q_ref[0, 0:1, :] * (1.0 / N_NODES)
    inv_std = lax.rsqrt(jnp.maximum(msq - m * m, 0.0) + 1e-5)
    c = c_ref[:, 0:1]
    dinv = jnp.where(c > 0, 1.0 / c, 0.0)
    y = _load_raw(r_ref, parts) * dinv
    return jnp.maximum((y - m) * inv_std * g_ref[0] + bt_ref[0], 0.0)


def _bn_mm(raw, cnt, g, bt, s, q, wc, parts, x0c=None):
    nci, nco = wc.shape[0], wc.shape[1]
    has_res = x0c is not None

    def body(*refs):
        if has_res:
            r_ref, c_ref, g_ref, bt_ref, s_ref, q_ref, x0_ref, w_ref, o_ref = refs
        else:
            r_ref, c_ref, g_ref, bt_ref, s_ref, q_ref, w_ref, o_ref = refs
        kk = pl.program_id(2)
        z = _bn_z(r_ref, c_ref, g_ref, bt_ref, s_ref, q_ref, parts)
        if has_res:
            z = z + x0_ref[0]
        acc = jnp.dot(z, w_ref[0, 0], preferred_element_type=jnp.float32)[None]

        @pl.when(kk == 0)
        def _():
            o_ref[...] = acc

        @pl.when(kk != 0)
        def _():
            o_ref[...] += acc

    in_specs = [
        _raw_spec(parts, lambda o, r, kk: (kk, r, 0)),
        pl.BlockSpec((BN, C), lambda o, r, kk: (r, 0)),
        pl.BlockSpec((1, 1, C), lambda o, r, kk: (kk, 0, 0)),
        pl.BlockSpec((1, 1, C), lambda o, r, kk: (kk, 0, 0)),
        pl.BlockSpec((1, 8, C), lambda o, r, kk: (kk, 0, 0)),
        pl.BlockSpec((1, 8, C), lambda o, r, kk: (kk, 0, 0)),
    ]
    args = [raw, cnt, g.reshape(nci, 1, C), bt.reshape(nci, 1, C), s, q]
    if has_res:
        in_specs.append(pl.BlockSpec((1, BN, C), lambda o, r, kk: (kk, r, 0)))
        args.append(x0c)
    in_specs.append(pl.BlockSpec((1, 1, C, C), lambda o, r, kk: (kk, o, 0, 0)))
    args.append(wc)

    return pl.pallas_call(
        body,
        grid=(nco, N_NODES // BN, nci),
        in_specs=in_specs,
        out_specs=pl.BlockSpec((1, BN, C), lambda o, r, kk: (o, r, 0)),
        out_shape=jax.ShapeDtypeStruct((nco, N_NODES, C), jnp.float32),
    )(*args)


def _bn_final(raw, cnt, g, bt, s, q, parts):
    nc = raw.shape[1] if parts else raw.shape[0]

    def body(r_ref, c_ref, g_ref, bt_ref, s_ref, q_ref, o_ref):
        o_ref[...] = _bn_z(r_ref, c_ref, g_ref, bt_ref, s_ref, q_ref, parts)[None]

    return pl.pallas_call(
        body,
        grid=(nc, N_NODES // BN),
        in_specs=[
            _raw_spec(parts, lambda o, r: (o, r, 0)),
            pl.BlockSpec((BN, C), lambda o, r: (r, 0)),
            pl.BlockSpec((1, 1, C), lambda o, r: (o, 0, 0)),
            pl.BlockSpec((1, 1, C), lambda o, r: (o, 0, 0)),
            pl.BlockSpec((1, 8, C), lambda o, r: (o, 0, 0)),
            pl.BlockSpec((1, 8, C), lambda o, r: (o, 0, 0)),
        ],
        out_specs=pl.BlockSpec((1, BN, C), lambda o, r: (o, r, 0)),
        out_shape=jax.ShapeDtypeStruct((nc, N_NODES, C), jnp.float32),
    )(raw, cnt, g.reshape(nc, 1, C), bt.reshape(nc, 1, C), s, q)


def _chunk_w(w):
    di, do = w.shape
    nci, nco = di // C, do // C
    return w.reshape(nci, C, nco, C).transpose(0, 2, 1, 3)


def kernel(x, edge, W1, b1, g1, bt1, W2, b2, g2, bt2, W3, b3, g3, bt3,
           W4, b4, g4, bt4):
    nidx = edge[0]
    hidx = edge[1]

    nid_s = _tile_pad(nidx, SENT)
    hid_s = _tile_pad(hidx, SENT)
    nid_g = _make_gather_idx(_tile_pad(nidx, 0), 3)
    hid_g = _make_gather_idx(_tile_pad(hidx, 0), 3)
    nid_s2 = _tile_pad2(nidx, SENT)
    hid_s2 = _tile_pad2(hidx, SENT)
    nid_g2 = _make_gather_idx2(_tile_pad2(nidx, 0), 3)
    hid_g2 = _make_gather_idx2(_tile_pad2(hidx, 0), 3)

    ones = jnp.ones((128, C), jnp.float32)
    zeros = jnp.zeros((632, C), jnp.float32)

    sidx2 = jnp.stack([nid_s, hid_s])
    cnts = _sc_degrees()(sidx2, ones, zeros)
    d16 = cnts[0]
    bd16 = cnts[1]

    pass_full = _sc_pass(True)
    pass_half = _sc_pass(False)

    def conv6(xw):
        he_raw = pass_full(xw.reshape(6 * N_NODES, C), nid_g, hid_s, zeros)
        he_s = _scale_rows(he_raw, bd16, False)
        return pass_full(he_s.reshape(6 * N_NODES, C), hid_g, nid_s, zeros)

    def conv3(xw):
        x2 = jnp.concatenate([xw.reshape(3 * N_NODES, C)] * 2)
        he_raw = pass_half(x2, nid_g2, hid_s2, zeros)
        he_s = _scale_rows(he_raw, bd16, True)
        h2 = jnp.concatenate([he_s.reshape(3 * N_NODES, C)] * 2)
        return pass_half(h2, hid_g2, nid_s2, zeros)

    xw = _mm_in_flat(x, W1, 6)
    r1 = conv6(xw)
    s1, q1 = _stats(r1, d16, False)
    xw = _bn_mm(r1, d16, g1, bt1, s1, q1, _chunk_w(W2), False)
    r2 = conv6(xw)
    s2, q2 = _stats(r2, d16, False)
    xw = _bn_mm(r2, d16, g2, bt2, s2, q2, _chunk_w(W3), False)
    r3 = conv3(xw)
    s3, q3 = _stats(r3, d16, True)
    x0c = x.reshape(N_NODES, 3, C).transpose(1, 0, 2)
    xw = _bn_mm(r3, d16, g3, bt3, s3, q3, _chunk_w(W4), True, x0c=x0c)
    r4 = conv3(xw)
    s4, q4 = _stats(r4, d16, True)
    h = _bn_final(r4, d16, g4, bt4, s4, q4, True)
    return h.transpose(1, 0, 2).reshape(N_NODES, 3 * C)


# speedup vs baseline: 1.8496x; 1.0009x over previous
import functools

import jax
import jax.numpy as jnp
from jax import lax
from jax.experimental import pallas as pl
from jax.experimental.pallas import tpu as pltpu
from jax.experimental.pallas import tpu_sc as plsc

N_NODES = 10000
N_HE = 10000
N_EDGES = 160000
NT = 16
NCORE = 2
C = 128
EPT = N_EDGES // NT
ECH = 80
EPAD = ECH * 128 - EPT
EPT2 = N_EDGES // (NCORE * NT)
ECH2 = 40
EPAD2 = ECH2 * 128 - EPT2
ACC_ROWS = 10112
SENT = N_NODES
BN = 1000


def _tile_pad(idx, sentinel):
    a = idx.reshape(NT, EPT)
    a = jnp.pad(a, ((0, 0), (0, EPAD)), constant_values=sentinel)
    return a.reshape(NT, ECH, 128)


def _make_gather_idx(idx_pad, nch):
    nc = NCORE * nch
    shift = (jnp.arange(nc, dtype=jnp.int32) * N_NODES)[:, None, None, None]
    g = idx_pad[None] + shift
    g = g.reshape(nc, NT, 2, 40, 128)
    g = jnp.pad(g, ((0, 0), (0, 0), (0, 0), (0, 1), (0, 0)))
    g = g.reshape(NCORE, nch, NT, 2, 41, 128)
    return g.transpose(0, 2, 1, 3, 4, 5)


def _tile_pad2(idx, sentinel):
    a = idx.reshape(NCORE, NT, EPT2)
    a = jnp.pad(a, ((0, 0), (0, 0), (0, EPAD2)), constant_values=sentinel)
    return a.reshape(NCORE, NT, ECH2, 128)


def _make_gather_idx2(idx_pad2, nch):
    shift = (jnp.arange(nch, dtype=jnp.int32) * N_NODES).reshape(1, 1, nch, 1, 1)
    cshift = (jnp.arange(NCORE, dtype=jnp.int32) * (nch * N_NODES)
              ).reshape(NCORE, 1, 1, 1, 1)
    g = idx_pad2[:, :, None] + shift + cshift
    g = jnp.pad(g, ((0, 0), (0, 0), (0, 0), (0, 1), (0, 0)))
    return g[:, :, :, None]


def _sc_pass(full):
    nh = 2 if full else 1
    mesh = plsc.VectorSubcoreMesh(core_axis_name="c", subcore_axis_name="s")
    out_t = (jax.ShapeDtypeStruct((6, N_NODES, C), jnp.float32) if full
             else jax.ShapeDtypeStruct((NCORE, 3, N_NODES, C), jnp.float32))

    @functools.partial(
        pl.kernel, mesh=mesh,
        out_type=out_t,
        scratch_types=[
            pltpu.VMEM((41, 128), jnp.int32),
            pltpu.VMEM((nh * 40, 128), jnp.int32),
            pltpu.VMEM((128, C), jnp.float32),
            pltpu.VMEM((128, C), jnp.float32),
            pltpu.VMEM_SHARED((ACC_ROWS, C), jnp.float32),
            pltpu.SemaphoreType.DMA,
            pltpu.SemaphoreType.DMA,
            pltpu.SemaphoreType.DMA,
        ],
    )
    def k(src, gidx, sidx, zeros, out, gidx_v, sidx_v, rows_a, rows_b,
          acc, sem_a, sem_b, sem_s):
        core = lax.axis_index("c")
        s = lax.axis_index("s")
        if full:
            pltpu.sync_copy(sidx.at[s], sidx_v)
        else:
            pltpu.sync_copy(sidx.at[core, s], sidx_v)
        for cc in range(3):
            pltpu.sync_copy(zeros, acc.at[pl.ds(s * 632, 632)])
            plsc.subcore_barrier()
            for h in range(nh):
                pltpu.sync_copy(gidx.at[core, s, cc, h], gidx_v)

                def drain():
                    pltpu.make_async_copy(rows_a, acc.at[sidx_v.at[0]],
                                          sem_s).wait()

                def pair(i, carry):
                    ja = 2 * i

                    @pl.when(i > 0)
                    def _():
                        drain()

                    pltpu.async_copy(src.at[gidx_v.at[ja]], rows_a,
                                     sem_a).wait()
                    pltpu.async_copy(rows_a, acc.at[sidx_v.at[h * 40 + ja]],
                                     sem_s, add=True)

                    @pl.when(i > 0)
                    def _():
                        drain()

                    pltpu.async_copy(src.at[gidx_v.at[ja + 1]], rows_b,
                                     sem_b).wait()
                    pltpu.async_copy(rows_b,
                                     acc.at[sidx_v.at[h * 40 + ja + 1]],
                                     sem_s, add=True)
                    return carry

                lax.fori_loop(0, 20, pair, 0)
                drain()
                drain()
            plsc.subcore_barrier()
            if full:
                chunk = core * 3 + cc
                pltpu.sync_copy(acc.at[pl.ds(s * 624, 640)],
                                out.at[chunk, pl.ds(s * 624, 640)])
            else:
                pltpu.sync_copy(acc.at[pl.ds(s * 624, 640)],
                                out.at[core, cc, pl.ds(s * 624, 640)])
            plsc.subcore_barrier()

    return k


def _sc_degrees():
    mesh = plsc.VectorSubcoreMesh(core_axis_name="c", subcore_axis_name="s")

    @functools.partial(
        pl.kernel, mesh=mesh,
        out_type=jax.ShapeDtypeStruct((2, N_NODES, C), jnp.float32),
        scratch_types=[
            pltpu.VMEM((ECH, 128), jnp.int32),
            pltpu.VMEM((128, C), jnp.float32),
            pltpu.VMEM_SHARED((ACC_ROWS, C), jnp.float32),
            pltpu.SemaphoreType.DMA,
        ],
    )
    def k(sidx2, ones, zeros, out, sidx_v, ones_v, acc, sem):
        core = lax.axis_index("c")
        s = lax.axis_index("s")
        pltpu.sync_copy(sidx2.at[core, s], sidx_v)
        pltpu.sync_copy(ones, ones_v)
        pltpu.sync_copy(zeros, acc.at[pl.ds(s * 632, 632)])
        plsc.subcore_barrier()

        def body(j, carry):
            pltpu.async_copy(ones_v, acc.at[sidx_v.at[j]], sem, add=True)
            return carry

        lax.fori_loop(0, ECH, body, 0)

        def dr(j, carry):
            pltpu.make_async_copy(ones_v, acc.at[sidx_v.at[0]], sem).wait()
            return carry

        lax.fori_loop(0, ECH, dr, 0)
        plsc.subcore_barrier()
        pltpu.sync_copy(acc.at[pl.ds(s * 624, 640)],
                        out.at[core, pl.ds(s * 624, 640)])

    return k


def _load_raw(r_ref, parts):
    return (r_ref[0, 0] + r_ref[1, 0]) if parts else r_ref[0]


def _raw_spec(parts, imap3):
    if parts:
        return pl.BlockSpec((2, 1, BN, C), lambda *g: (0,) + imap3(*g))
    return pl.BlockSpec((1, BN, C), imap3)


def _mm_in_flat(x, w, nco):
    din = x.shape[1]

    def body(x_ref, w_ref, o_ref):
        o_ref[...] = jnp.dot(x_ref[...], w_ref[...],
                             preferred_element_type=jnp.float32)[None]

    return pl.pallas_call(
        body,
        grid=(nco, N_NODES // BN),
        in_specs=[
            pl.BlockSpec((BN, din), lambda o, r: (r, 0)),
            pl.BlockSpec((din, C), lambda o, r: (0, o)),
        ],
        out_specs=pl.BlockSpec((1, BN, C), lambda o, r: (o, r, 0)),
        out_shape=jax.ShapeDtypeStruct((nco, N_NODES, C), jnp.float32),
    )(x, w)


def _scale_rows(raw, cnt, parts):
    nc = raw.shape[1] if parts else raw.shape[0]

    def body(r_ref, c_ref, o_ref):
        c = c_ref[:, 0:1]
        inv = jnp.where(c > 0, 1.0 / c, 0.0)
        o_ref[...] = (_load_raw(r_ref, parts) * inv)[None]

    return pl.pallas_call(
        body,
        grid=(nc, N_NODES // BN),
        in_specs=[
            _raw_spec(parts, lambda o, r: (o, r, 0)),
            pl.BlockSpec((BN, C), lambda o, r: (r, 0)),
        ],
        out_specs=pl.BlockSpec((1, BN, C), lambda o, r: (o, r, 0)),
        out_shape=jax.ShapeDtypeStruct((nc, N_NODES, C), jnp.float32),
    )(raw, cnt)


def _stats(raw, cnt, parts):
    nc = raw.shape[1] if parts else raw.shape[0]

    def body(r_ref, c_ref, s_ref, q_ref):
        r = pl.program_id(1)
        c = c_ref[:, 0:1]
        inv = jnp.where(c > 0, 1.0 / c, 0.0)
        y = _load_raw(r_ref, parts) * inv
        s1 = jnp.broadcast_to(jnp.sum(y, axis=0, keepdims=True), (8, C))[None]
        q1 = jnp.broadcast_to(jnp.sum(y * y, axis=0, keepdims=True), (8, C))[None]

        @pl.when(r == 0)
        def _():
            s_ref[...] = s1
            q_ref[...] = q1

        @pl.when(r != 0)
        def _():
            s_ref[...] += s1
            q_ref[...] += q1

    return pl.pallas_call(
        body,
        grid=(nc, N_NODES // BN),
        in_specs=[
            _raw_spec(parts, lambda o, r: (o, r, 0)),
            pl.BlockSpec((BN, C), lambda o, r: (r, 0)),
        ],
        out_specs=[
            pl.BlockSpec((1, 8, C), lambda o, r: (o, 0, 0)),
            pl.BlockSpec((1, 8, C), lambda o, r: (o, 0, 0)),
        ],
        out_shape=[
            jax.ShapeDtypeStruct((nc, 8, C), jnp.float32),
            jax.ShapeDtypeStruct((nc, 8, C), jnp.float32),
        ],
    )(raw, cnt)


def _bn_z(r_ref, c_ref, g_ref, bt_ref, s_ref, q_ref, parts):
    m = s_ref[0, 0:1, :] * (1.0 / N_NODES)
    msq = q_ref[0, 0:1, :] * (1.0 / N_NODES)
    inv_std = lax.rsqrt(jnp.maximum(msq - m * m, 0.0) + 1e-5)
    c = c_ref[:, 0:1]
    dinv = jnp.where(c > 0, 1.0 / c, 0.0)
    y = _load_raw(r_ref, parts) * dinv
    return jnp.maximum((y - m) * inv_std * g_ref[0] + bt_ref[0], 0.0)


def _bn_mm(raw, cnt, g, bt, s, q, wc, parts, x0c=None):
    nci, nco = wc.shape[0], wc.shape[1]
    has_res = x0c is not None

    def body(*refs):
        if has_res:
            r_ref, c_ref, g_ref, bt_ref, s_ref, q_ref, x0_ref, w_ref, o_ref = refs
        else:
            r_ref, c_ref, g_ref, bt_ref, s_ref, q_ref, w_ref, o_ref = refs
        kk = pl.program_id(2)
        z = _bn_z(r_ref, c_ref, g_ref, bt_ref, s_ref, q_ref, parts)
        if has_res:
            z = z + x0_ref[0]
        acc = jnp.dot(z, w_ref[0, 0], preferred_element_type=jnp.float32)[None]

        @pl.when(kk == 0)
        def _():
            o_ref[...] = acc

        @pl.when(kk != 0)
        def _():
            o_ref[...] += acc

    in_specs = [
        _raw_spec(parts, lambda o, r, kk: (kk, r, 0)),
        pl.BlockSpec((BN, C), lambda o, r, kk: (r, 0)),
        pl.BlockSpec((1, 1, C), lambda o, r, kk: (kk, 0, 0)),
        pl.BlockSpec((1, 1, C), lambda o, r, kk: (kk, 0, 0)),
        pl.BlockSpec((1, 8, C), lambda o, r, kk: (kk, 0, 0)),
        pl.BlockSpec((1, 8, C), lambda o, r, kk: (kk, 0, 0)),
    ]
    args = [raw, cnt, g.reshape(nci, 1, C), bt.reshape(nci, 1, C), s, q]
    if has_res:
        in_specs.append(pl.BlockSpec((1, BN, C), lambda o, r, kk: (kk, r, 0)))
        args.append(x0c)
    in_specs.append(pl.BlockSpec((1, 1, C, C), lambda o, r, kk: (kk, o, 0, 0)))
    args.append(wc)

    return pl.pallas_call(
        body,
        grid=(nco, N_NODES // BN, nci),
        in_specs=in_specs,
        out_specs=pl.BlockSpec((1, BN, C), lambda o, r, kk: (o, r, 0)),
        out_shape=jax.ShapeDtypeStruct((nco, N_NODES, C), jnp.float32),
    )(*args)


def _bn_final(raw, cnt, g, bt, s, q, parts):
    nc = raw.shape[1] if parts else raw.shape[0]

    def body(r_ref, c_ref, g_ref, bt_ref, s_ref, q_ref, o_ref):
        o_ref[...] = _bn_z(r_ref, c_ref, g_ref, bt_ref, s_ref, q_ref, parts)[None]

    return pl.pallas_call(
        body,
        grid=(nc, N_NODES // BN),
        in_specs=[
            _raw_spec(parts, lambda o, r: (o, r, 0)),
            pl.BlockSpec((BN, C), lambda o, r: (r, 0)),
            pl.BlockSpec((1, 1, C), lambda o, r: (o, 0, 0)),
            pl.BlockSpec((1, 1, C), lambda o, r: (o, 0, 0)),
            pl.BlockSpec((1, 8, C), lambda o, r: (o, 0, 0)),
            pl.BlockSpec((1, 8, C), lambda o, r: (o, 0, 0)),
        ],
        out_specs=pl.BlockSpec((1, BN, C), lambda o, r: (o, r, 0)),
        out_shape=jax.ShapeDtypeStruct((nc, N_NODES, C), jnp.float32),
    )(raw, cnt, g.reshape(nc, 1, C), bt.reshape(nc, 1, C), s, q)


def _chunk_w(w):
    di, do = w.shape
    nci, nco = di // C, do // C
    return w.reshape(nci, C, nco, C).transpose(0, 2, 1, 3)


def kernel(x, edge, W1, b1, g1, bt1, W2, b2, g2, bt2, W3, b3, g3, bt3,
           W4, b4, g4, bt4):
    nidx = edge[0]
    hidx = edge[1]

    nid_s = _tile_pad(nidx, SENT)
    hid_s = _tile_pad(hidx, SENT)
    nid_g = _make_gather_idx(_tile_pad(nidx, 0), 3)
    hid_g = _make_gather_idx(_tile_pad(hidx, 0), 3)
    nid_s2 = _tile_pad2(nidx, SENT)
    hid_s2 = _tile_pad2(hidx, SENT)
    nid_g2 = _make_gather_idx2(_tile_pad2(nidx, 0), 3)
    hid_g2 = _make_gather_idx2(_tile_pad2(hidx, 0), 3)

    ones = jnp.ones((128, C), jnp.float32)
    zeros = jnp.zeros((632, C), jnp.float32)

    sidx2 = jnp.stack([nid_s, hid_s])
    cnts = _sc_degrees()(sidx2, ones, zeros)
    d16 = cnts[0]
    bd16 = cnts[1]

    pass_full = _sc_pass(True)
    pass_half = _sc_pass(False)

    def conv6(xw):
        he_raw = pass_full(xw.reshape(6 * N_NODES, C), nid_g, hid_s, zeros)
        he_s = _scale_rows(he_raw, bd16, False)
        return pass_full(he_s.reshape(6 * N_NODES, C), hid_g, nid_s, zeros)

    def conv3(xw):
        x2 = jnp.concatenate([xw.reshape(3 * N_NODES, C)] * 2)
        he_raw = pass_half(x2, nid_g2, hid_s2, zeros)
        he_s = _scale_rows(he_raw, bd16, True)
        h2 = jnp.concatenate([he_s.reshape(3 * N_NODES, C)] * 2)
        return pass_half(h2, hid_g2, nid_s2, zeros)

    xw = _mm_in_flat(x, W1, 6)
    r1 = conv6(xw)
    s1, q1 = _stats(r1, d16, False)
    xw = _bn_mm(r1, d16, g1, bt1, s1, q1, _chunk_w(W2), False)
    r2 = conv6(xw)
    s2, q2 = _stats(r2, d16, False)
    xw = _bn_mm(r2, d16, g2, bt2, s2, q2, _chunk_w(W3), False)
    r3 = conv3(xw)
    s3, q3 = _stats(r3, d16, True)
    x0c = x.reshape(N_NODES, 3, C).transpose(1, 0, 2)
    xw = _bn_mm(r3, d16, g3, bt3, s3, q3, _chunk_w(W4), True, x0c=x0c)
    r4 = conv3(xw)
    s4, q4 = _stats(r4, d16, True)
    h = _bn_final(r4, d16, g4, bt4, s4, q4, True)
    return h.transpose(1, 0, 2).reshape(N_NODES, 3 * C)
